# Initial kernel scaffold; baseline (speedup 1.0000x reference)
import jax
import jax.numpy as jnp
from jax.experimental import pallas as pl


def kernel(features, Wl1, bl1, Wr1, Wl2, bl2, Wr2, Wih0, Whh0, bih0, bhh0, Wih1, Whh1, bih1, bhh1, fc_w, fc_b):
    raise NotImplementedError("write your pallas kernel here")


# speedup vs baseline: 8.5665x; 8.5665x over previous
import jax
import jax.numpy as jnp
from jax.experimental import pallas as pl
from jax.experimental.pallas import tpu as pltpu

H = 128
W = 20
B = 256
HOR = 12
OUTP = 128


def _body(x0_ref, wr1_ref, bl1_ref, wr2_ref, bl2_ref,
          wih0_ref, whh0_ref, bih0_ref, bhh0_ref,
          wih1_ref, whh1_ref, bih1_ref, bhh1_ref,
          wfc_ref, fcb_ref, out_ref, seq_ref, gi_ref):
    f32 = jnp.float32

    w12 = jnp.dot(wr1_ref[:], wr2_ref[:], preferred_element_type=f32)
    b12 = jnp.dot(bl1_ref[:], wr2_ref[:], preferred_element_type=f32) + bl2_ref[:]
    seq_ref[:] = jnp.dot(x0_ref[:], w12, preferred_element_type=f32) + b12

    h = [jnp.zeros((B, H), f32), jnp.zeros((B, H), f32)]
    layers = ((wih0_ref, whh0_ref, bih0_ref, bhh0_ref),
              (wih1_ref, whh1_ref, bih1_ref, bhh1_ref))
    out_acc = jnp.zeros((B, OUTP), f32) + fcb_ref[:]

    for k in range(HOR):
        for l in range(2):
            wih, whh, bih, bhh = layers[l]
            gi_ref[:] = (jnp.dot(seq_ref[:], wih[:], preferred_element_type=f32)
                         + bih[:])

            def step(t, hc, _whh=whh, _bhh=bhh):
                gh = jnp.dot(hc, _whh[:], preferred_element_type=f32) + _bhh[:]
                gi_t = gi_ref[pl.ds(t * B, B), :]
                r = jax.nn.sigmoid(gi_t[:, :H] + gh[:, :H])
                z = jax.nn.sigmoid(gi_t[:, H:2 * H] + gh[:, H:2 * H])
                n = jnp.tanh(gi_t[:, 2 * H:] + r * gh[:, 2 * H:])
                hn = (1.0 - z) * n + z * hc
                seq_ref[pl.ds(t * B, B), :] = hn
                return hn

            h[l] = jax.lax.fori_loop(0, W, step, h[l])
        out_acc = out_acc + jnp.dot(h[1], wfc_ref[k * H:(k + 1) * H, :],
                                    preferred_element_type=f32)
    out_ref[:] = out_acc


def kernel(features, Wl1, bl1, Wr1, Wl2, bl2, Wr2, Wih0, Whh0, bih0, bhh0,
           Wih1, Whh1, bih1, bhh1, fc_w, fc_b):
    f32 = jnp.float32
    x0 = jnp.transpose(features[:, :, 0, :], (1, 0, 2)).reshape(W * B, H)
    wfc = jnp.kron(jnp.eye(HOR, OUTP, dtype=f32), fc_w.reshape(H, 1))
    fcb = jnp.broadcast_to(fc_b.reshape(1, 1), (1, OUTP))

    out = pl.pallas_call(
        _body,
        out_shape=jax.ShapeDtypeStruct((B, OUTP), f32),
        scratch_shapes=[
            pltpu.VMEM((W * B, H), f32),
            pltpu.VMEM((W * B, 3 * H), f32),
        ],
    )(x0, Wr1, bl1.reshape(1, H), Wr2, bl2.reshape(1, H),
      Wih0.T, Whh0.T, bih0.reshape(1, 3 * H), bhh0.reshape(1, 3 * H),
      Wih1.T, Whh1.T, bih1.reshape(1, 3 * H), bhh1.reshape(1, 3 * H),
      wfc, fcb)
    return out[:, :HOR]


# speedup vs baseline: 8.6901x; 1.0144x over previous
import jax
import jax.numpy as jnp
from jax.experimental import pallas as pl
from jax.experimental.pallas import tpu as pltpu

H = 128
W = 20
B = 256
HOR = 12
OUTP = 128


def _body(x0_ref, wr1_ref, bl1_ref, wr2_ref, bl2_ref,
          wih0_ref, whh0_ref, bih0_ref, bhh0_ref,
          wih1_ref, whh1_ref, bih1_ref, bhh1_ref,
          wfc_ref, fcb_ref, out_ref, seq_ref, gi_ref):
    f32 = jnp.float32
    bf16 = jnp.bfloat16

    w12 = jnp.dot(wr1_ref[:], wr2_ref[:], preferred_element_type=f32)
    b12 = jnp.dot(bl1_ref[:], wr2_ref[:], preferred_element_type=f32) + bl2_ref[:]
    seq_ref[:] = jnp.dot(x0_ref[:], w12, preferred_element_type=f32) + b12

    h = [jnp.zeros((B, H), f32), jnp.zeros((B, H), f32)]
    layers = ((wih0_ref, whh0_ref, bih0_ref, bhh0_ref),
              (wih1_ref, whh1_ref, bih1_ref, bhh1_ref))
    out_acc = jnp.zeros((B, OUTP), f32) + fcb_ref[:]

    for k in range(HOR):
        for l in range(2):
            wih, whh, bih, bhh = layers[l]
            gi_ref[:] = (jnp.dot(seq_ref[:], wih[:], preferred_element_type=f32)
                         + bih[:])

            def step(t, hc, _whh=whh, _bhh=bhh):
                gh = jnp.dot(hc.astype(bf16), _whh[:],
                             preferred_element_type=f32) + _bhh[:]
                gi_t = gi_ref[pl.ds(t * B, B), :]
                r = jax.nn.sigmoid(gi_t[:, :H] + gh[:, :H])
                z = jax.nn.sigmoid(gi_t[:, H:2 * H] + gh[:, H:2 * H])
                n = jnp.tanh(gi_t[:, 2 * H:] + r * gh[:, 2 * H:])
                hn = (1.0 - z) * n + z * hc
                seq_ref[pl.ds(t * B, B), :] = hn
                return hn

            h[l] = jax.lax.fori_loop(0, W, step, h[l])
        out_acc = out_acc + jnp.dot(h[1], wfc_ref[k * H:(k + 1) * H, :],
                                    preferred_element_type=f32)
    out_ref[:] = out_acc


def kernel(features, Wl1, bl1, Wr1, Wl2, bl2, Wr2, Wih0, Whh0, bih0, bhh0,
           Wih1, Whh1, bih1, bhh1, fc_w, fc_b):
    f32 = jnp.float32
    x0 = jnp.transpose(features[:, :, 0, :], (1, 0, 2)).reshape(W * B, H)
    wfc = jnp.kron(jnp.eye(HOR, OUTP, dtype=f32), fc_w.reshape(H, 1))
    fcb = jnp.broadcast_to(fc_b.reshape(1, 1), (1, OUTP))

    out = pl.pallas_call(
        _body,
        out_shape=jax.ShapeDtypeStruct((B, OUTP), f32),
        scratch_shapes=[
            pltpu.VMEM((W * B, H), f32),
            pltpu.VMEM((W * B, 3 * H), f32),
        ],
    )(x0, Wr1, bl1.reshape(1, H), Wr2, bl2.reshape(1, H),
      Wih0.T, Whh0.T.astype(jnp.bfloat16), bih0.reshape(1, 3 * H),
      bhh0.reshape(1, 3 * H),
      Wih1.T, Whh1.T.astype(jnp.bfloat16), bih1.reshape(1, 3 * H),
      bhh1.reshape(1, 3 * H),
      wfc, fcb)
    return out[:, :HOR]


# speedup vs baseline: 12.0429x; 1.3858x over previous
import jax
import jax.numpy as jnp
from jax.experimental import pallas as pl
from jax.experimental.pallas import tpu as pltpu

H = 128
W = 20
B = 256
HOR = 12
OUTP = 128


def _cell(gi, gh, hc):
    r = jax.nn.sigmoid(gi[:, :H] + gh[:, :H])
    z = jax.nn.sigmoid(gi[:, H:2 * H] + gh[:, H:2 * H])
    n = jnp.tanh(gi[:, 2 * H:] + r * gh[:, 2 * H:])
    return (1.0 - z) * n + z * hc


def _body(x0_ref, wr1_ref, bl1_ref, wr2_ref, bl2_ref,
          wih0_ref, whh0_ref, bih0_ref, bhh0_ref,
          wih1_ref, whh1_ref, bih1_ref, bhh1_ref,
          wfc_ref, fcb_ref, out_ref, seq_ref, gi_ref):
    f32 = jnp.float32
    bf16 = jnp.bfloat16

    def ghh(hc, whh_ref, bhh_ref):
        return jnp.dot(hc.astype(bf16), whh_ref[:],
                       preferred_element_type=f32) + bhh_ref[:]

    def gih(hc, wih_ref, bih_ref):
        return jnp.dot(hc, wih_ref[:], preferred_element_type=f32) + bih_ref[:]

    w12 = jnp.dot(wr1_ref[:], wr2_ref[:], preferred_element_type=f32)
    b12 = jnp.dot(bl1_ref[:], wr2_ref[:], preferred_element_type=f32) + bl2_ref[:]
    seq_ref[:] = jnp.dot(x0_ref[:], w12, preferred_element_type=f32) + b12

    h0 = jnp.zeros((B, H), f32)
    h1 = jnp.zeros((B, H), f32)
    out_acc = jnp.zeros((B, OUTP), f32) + fcb_ref[:]

    for k in range(HOR):
        gi_ref[:] = (jnp.dot(seq_ref[:], wih0_ref[:], preferred_element_type=f32)
                     + bih0_ref[:])

        h0 = _cell(gi_ref[pl.ds(0, B), :], ghh(h0, whh0_ref, bhh0_ref), h0)

        def body(t, carry):
            hc0, hc1, h0d = carry
            hn0 = _cell(gi_ref[pl.ds(t * B, B), :],
                        ghh(hc0, whh0_ref, bhh0_ref), hc0)
            hn1 = _cell(gih(h0d, wih1_ref, bih1_ref),
                        ghh(hc1, whh1_ref, bhh1_ref), hc1)
            seq_ref[pl.ds((t - 1) * B, B), :] = hn1
            return hn0, hn1, hn0

        h0, h1, h0d = jax.lax.fori_loop(1, W, body, (h0, h1, h0))

        h1 = _cell(gih(h0d, wih1_ref, bih1_ref),
                   ghh(h1, whh1_ref, bhh1_ref), h1)
        seq_ref[pl.ds((W - 1) * B, B), :] = h1

        out_acc = out_acc + jnp.dot(h1, wfc_ref[k * H:(k + 1) * H, :],
                                    preferred_element_type=f32)
    out_ref[:] = out_acc


def kernel(features, Wl1, bl1, Wr1, Wl2, bl2, Wr2, Wih0, Whh0, bih0, bhh0,
           Wih1, Whh1, bih1, bhh1, fc_w, fc_b):
    f32 = jnp.float32
    x0 = jnp.transpose(features[:, :, 0, :], (1, 0, 2)).reshape(W * B, H)
    wfc = jnp.kron(jnp.eye(HOR, OUTP, dtype=f32), fc_w.reshape(H, 1))
    fcb = jnp.broadcast_to(fc_b.reshape(1, 1), (1, OUTP))

    out = pl.pallas_call(
        _body,
        out_shape=jax.ShapeDtypeStruct((B, OUTP), f32),
        scratch_shapes=[
            pltpu.VMEM((W * B, H), f32),
            pltpu.VMEM((W * B, 3 * H), f32),
        ],
    )(x0, Wr1, bl1.reshape(1, H), Wr2, bl2.reshape(1, H),
      Wih0.T, Whh0.T.astype(jnp.bfloat16), bih0.reshape(1, 3 * H),
      bhh0.reshape(1, 3 * H),
      Wih1.T, Whh1.T.astype(jnp.bfloat16), bih1.reshape(1, 3 * H),
      bhh1.reshape(1, 3 * H),
      wfc, fcb)
    return out[:, :HOR]
